# baseline (speedup 1.0000x reference)
import functools

import jax
import jax.numpy as jnp
from jax import lax
from jax.experimental import pallas as pl
from jax.experimental.pallas import tpu as pltpu
from jax.experimental.pallas import tpu_sc as plsc


def _transform_table(emb, w, b, blk2):
    R, E = emb.shape
    H = w.shape[1]
    R2 = R // 2
    n = R2 // blk2

    def body(lo_ref, hi_ref, w_ref, b_ref, out_ref):
        wv = w_ref[...]
        bv = b_ref[...]
        out_ref[:, 0:H] = (
            jnp.dot(lo_ref[...], wv, preferred_element_type=jnp.float32) + bv
        )
        out_ref[:, H : 2 * H] = (
            jnp.dot(hi_ref[...], wv, preferred_element_type=jnp.float32) + bv
        )

    return pl.pallas_call(
        body,
        grid=(n,),
        in_specs=[
            pl.BlockSpec((blk2, E), lambda i: (i, 0)),
            pl.BlockSpec((blk2, E), lambda i: (i + n, 0)),
            pl.BlockSpec((E, H), lambda i: (0, 0)),
            pl.BlockSpec((1, H), lambda i: (0, 0)),
        ],
        out_specs=pl.BlockSpec((blk2, 2 * H), lambda i: (i, 0)),
        out_shape=jax.ShapeDtypeStruct((R2, 2 * H), jnp.float32),
    )(emb, emb, w, b)


def _sc_lookup_sum(t1, t2, uid, sid, r1, r2, B, L, hdim, num_workers, rows_per_chunk):
    tok = B * L
    per_w = tok // num_workers
    rows_w = B // num_workers
    chunk = rows_per_chunk * L
    n_chunks = rows_w // rows_per_chunk
    mesh = plsc.VectorSubcoreMesh(core_axis_name="c", subcore_axis_name="s")
    nc = mesh.num_cores
    r1_half = r1 // 2
    r2_half = r2 // 2

    @functools.partial(
        pl.kernel,
        out_type=jax.ShapeDtypeStruct((B, L, hdim), jnp.float32),
        mesh=mesh,
        scratch_types=[
            pltpu.VMEM((chunk,), jnp.int32),
            pltpu.VMEM((chunk,), jnp.int32),
            pltpu.VMEM((chunk,), jnp.int32),
            pltpu.VMEM((chunk,), jnp.int32),
            pltpu.VMEM((chunk, hdim), jnp.float32),
            pltpu.VMEM((chunk, hdim), jnp.float32),
            pltpu.SemaphoreType.DMA,
            pltpu.SemaphoreType.DMA,
            pltpu.SemaphoreType.DMA,
            pltpu.SemaphoreType.DMA,
            pltpu.SemaphoreType.DMA,
            pltpu.SemaphoreType.DMA,
        ],
        compiler_params=pltpu.CompilerParams(use_tc_tiling_on_sc=False),
    )
    def k(
        t1_hbm,
        t2_hbm,
        uid_hbm,
        sid_hbm,
        out_hbm,
        idx1a,
        idx2a,
        idx1b,
        idx2b,
        bufa,
        bufb,
        sem_ga,
        sem_gb,
        sem_ga2,
        sem_gb2,
        sem_oa,
        sem_ob,
    ):
        wid = lax.axis_index("s") * nc + lax.axis_index("c")
        base = wid * per_w
        base_b = wid * rows_w
        slots = (
            (idx1a, idx2a, bufa, sem_ga, sem_ga2, sem_oa),
            (idx1b, idx2b, bufb, sem_gb, sem_gb2, sem_ob),
        )
        half = chunk // 2
        h0 = pl.ds(0, half)
        h1 = pl.ds(half, half)

        def stage_idx(i, idx1_v, idx2_v):
            off = base + i * chunk
            pltpu.sync_copy(uid_hbm.at[pl.ds(off, chunk)], idx1_v)
            pltpu.sync_copy(sid_hbm.at[pl.ds(off, chunk)], idx2_v)
            for kk in range(chunk // 16):
                sl = pl.ds(kk * 16, 16)
                v1 = idx1_v[sl]
                idx1_v[sl] = v1 + v1 - jnp.where(v1 < r1_half, 0, r1 - 1)
                v2 = idx2_v[sl]
                idx2_v[sl] = v2 + v2 - jnp.where(v2 < r2_half, 0, r2 - 1)

        def out_descs(i, buf, sem):
            b0 = base_b + i * rows_per_chunk
            return [
                pltpu.make_async_copy(buf.at[pl.ds(r * L, L)], out_hbm.at[b0 + r], sem)
                for r in range(rows_per_chunk)
            ]

        stage_idx(0, idx1a, idx2a)
        stage_idx(1, idx1b, idx2b)

        def pair_body(g, carry):
            for slot in (0, 1):
                idx1_v, idx2_v, buf, sem_g, sem_g2, sem_o = slots[slot]
                i = g * 2 + slot
                @pl.when(i >= 2)
                def _():
                    for d in out_descs(i - 2, buf, sem_o):
                        d.wait()

                c1a = pltpu.async_copy(t1_hbm.at[idx1_v.at[h0]], buf.at[h0], sem_g)
                c1b = pltpu.async_copy(t1_hbm.at[idx1_v.at[h1]], buf.at[h1], sem_g2)
                c1a.wait()
                c2a = pltpu.async_copy(
                    t2_hbm.at[idx2_v.at[h0]], buf.at[h0], sem_g, add=True
                )
                c1b.wait()
                c2b = pltpu.async_copy(
                    t2_hbm.at[idx2_v.at[h1]], buf.at[h1], sem_g2, add=True
                )
                c2a.wait()
                c2b.wait()
                for d in out_descs(i, buf, sem_o):
                    d.start()
                @pl.when(i + 2 < n_chunks)
                def _():
                    stage_idx(i + 2, idx1_v, idx2_v)
            return carry

        lax.fori_loop(0, n_chunks // 2, pair_body, 0)

        for slot in (0, 1):
            idx1_v, idx2_v, buf, sem_g, sem_g2, sem_o = slots[slot]
            for d in out_descs(n_chunks - 2 + slot, buf, sem_o):
                d.wait()

    return k(t1, t2, uid, sid)


def kernel(user_id_sequence, skill_sequence, emb1, emb2, W1, b1):
    B, L = user_id_sequence.shape
    E = emb1.shape[1]
    H = W1.shape[1]
    tok = B * L
    r1 = emb1.shape[0]
    r2 = emb2.shape[0]

    b_row = b1.reshape(1, H).astype(jnp.float32)
    zero_row = jnp.zeros((1, H), dtype=jnp.float32)
    t1 = _transform_table(emb1, W1[:E], b_row, blk2=10000).reshape(r1, H)
    t2 = _transform_table(emb2, W1[E:], zero_row, blk2=10000).reshape(r2, H)

    uid = user_id_sequence.reshape(tok).astype(jnp.int32)
    sid = skill_sequence.reshape(tok).astype(jnp.int32)

    return _sc_lookup_sum(
        t1, t2, uid, sid, r1, r2, B, L, H, num_workers=32, rows_per_chunk=4
    )


# speedup vs baseline: 1.0041x; 1.0041x over previous
import functools

import jax
import jax.numpy as jnp
from jax import lax
from jax.experimental import pallas as pl
from jax.experimental.pallas import tpu as pltpu
from jax.experimental.pallas import tpu_sc as plsc


def _transform_table(emb, w, b, blk2):
    R, E = emb.shape
    H = w.shape[1]
    R2 = R // 2
    n = R2 // blk2

    def body(lo_ref, hi_ref, w_ref, b_ref, out_ref):
        wv = w_ref[...]
        bv = b_ref[...]
        out_ref[:, 0:H] = (
            jnp.dot(lo_ref[...], wv, preferred_element_type=jnp.float32) + bv
        )
        out_ref[:, H : 2 * H] = (
            jnp.dot(hi_ref[...], wv, preferred_element_type=jnp.float32) + bv
        )

    return pl.pallas_call(
        body,
        grid=(n,),
        in_specs=[
            pl.BlockSpec((blk2, E), lambda i: (i, 0)),
            pl.BlockSpec((blk2, E), lambda i: (i + n, 0)),
            pl.BlockSpec((E, H), lambda i: (0, 0)),
            pl.BlockSpec((1, H), lambda i: (0, 0)),
        ],
        out_specs=pl.BlockSpec((blk2, 2 * H), lambda i: (i, 0)),
        out_shape=jax.ShapeDtypeStruct((R2, 2 * H), jnp.float32),
    )(emb, emb, w, b)


def _sc_lookup_sum(t1, t2, uid, sid, r1, r2, B, L, hdim, num_workers, rows_per_chunk):
    tok = B * L
    per_w = tok // num_workers
    rows_w = B // num_workers
    chunk = rows_per_chunk * L
    n_chunks = rows_w // rows_per_chunk
    mesh = plsc.VectorSubcoreMesh(core_axis_name="c", subcore_axis_name="s")
    nc = mesh.num_cores
    r1_half = r1 // 2
    r2_half = r2 // 2

    @functools.partial(
        pl.kernel,
        out_type=jax.ShapeDtypeStruct((B, L, hdim), jnp.float32),
        mesh=mesh,
        scratch_types=[
            pltpu.VMEM((chunk,), jnp.int32),
            pltpu.VMEM((chunk,), jnp.int32),
            pltpu.VMEM((chunk,), jnp.int32),
            pltpu.VMEM((chunk,), jnp.int32),
            pltpu.VMEM((chunk, hdim), jnp.float32),
            pltpu.VMEM((chunk, hdim), jnp.float32),
            pltpu.SemaphoreType.DMA,
            pltpu.SemaphoreType.DMA,
            pltpu.SemaphoreType.DMA,
            pltpu.SemaphoreType.DMA,
            pltpu.SemaphoreType.DMA,
            pltpu.SemaphoreType.DMA,
        ],
        compiler_params=pltpu.CompilerParams(use_tc_tiling_on_sc=False),
    )
    def k(
        t1_hbm,
        t2_hbm,
        uid_hbm,
        sid_hbm,
        out_hbm,
        idx1a,
        idx2a,
        idx1b,
        idx2b,
        bufa,
        bufb,
        sem_ga,
        sem_gb,
        sem_ga2,
        sem_gb2,
        sem_oa,
        sem_ob,
    ):
        wid = lax.axis_index("s") * nc + lax.axis_index("c")
        base = wid * per_w
        base_b = wid * rows_w
        slots = (
            (idx1a, idx2a, bufa, sem_ga, sem_ga2, sem_oa),
            (idx1b, idx2b, bufb, sem_gb, sem_gb2, sem_ob),
        )

        def stage_idx(i, idx1_v, idx2_v):
            off = base + i * chunk
            pltpu.sync_copy(uid_hbm.at[pl.ds(off, chunk)], idx1_v)
            pltpu.sync_copy(sid_hbm.at[pl.ds(off, chunk)], idx2_v)
            for kk in range(chunk // 16):
                sl = pl.ds(kk * 16, 16)
                v1 = idx1_v[sl]
                idx1_v[sl] = v1 + v1 - jnp.where(v1 < r1_half, 0, r1 - 1)
                v2 = idx2_v[sl]
                idx2_v[sl] = v2 + v2 - jnp.where(v2 < r2_half, 0, r2 - 1)

        def out_descs(i, buf, sem):
            b0 = base_b + i * rows_per_chunk
            return [
                pltpu.make_async_copy(buf.at[pl.ds(r * L, L)], out_hbm.at[b0 + r], sem)
                for r in range(rows_per_chunk)
            ]

        stage_idx(0, idx1a, idx2a)
        stage_idx(1, idx1b, idx2b)

        def pair_body(g, carry):
            for slot in (0, 1):
                idx1_v, idx2_v, buf, sem_g, sem_g2, sem_o = slots[slot]
                i = g * 2 + slot
                @pl.when(i >= 2)
                def _():
                    for d in out_descs(i - 2, buf, sem_o):
                        d.wait()

                pltpu.async_copy(t1_hbm.at[idx1_v], buf, sem_g).wait()
                pltpu.async_copy(t2_hbm.at[idx2_v], buf, sem_g, add=True).wait()
                for d in out_descs(i, buf, sem_o):
                    d.start()
                @pl.when(i + 2 < n_chunks)
                def _():
                    stage_idx(i + 2, idx1_v, idx2_v)
            return carry

        lax.fori_loop(0, n_chunks // 2, pair_body, 0)

        for slot in (0, 1):
            idx1_v, idx2_v, buf, sem_g, sem_g2, sem_o = slots[slot]
            for d in out_descs(n_chunks - 2 + slot, buf, sem_o):
                d.wait()

    return k(t1, t2, uid, sid)


def kernel(user_id_sequence, skill_sequence, emb1, emb2, W1, b1):
    B, L = user_id_sequence.shape
    E = emb1.shape[1]
    H = W1.shape[1]
    tok = B * L
    r1 = emb1.shape[0]
    r2 = emb2.shape[0]

    b_row = b1.reshape(1, H).astype(jnp.float32)
    zero_row = jnp.zeros((1, H), dtype=jnp.float32)
    t1 = _transform_table(emb1, W1[:E], b_row, blk2=10000).reshape(r1, H)
    t2 = _transform_table(emb2, W1[E:], zero_row, blk2=10000).reshape(r2, H)

    uid = user_id_sequence.reshape(tok).astype(jnp.int32)
    sid = skill_sequence.reshape(tok).astype(jnp.int32)

    return _sc_lookup_sum(
        t1, t2, uid, sid, r1, r2, B, L, H, num_workers=32, rows_per_chunk=4
    )
